# baseline (speedup 1.0000x reference)
import functools

import jax
import jax.numpy as jnp
from jax import lax
from jax.experimental import pallas as pl
from jax.experimental.pallas import tpu as pltpu
from jax.experimental.pallas import tpu_sc as plsc

N_ROWS = 8192
C = 2048
L = 16
NC = 2
NS = 16
NW = NC * NS
ROWS_PER_W = N_ROWS // NW
R = 16
CHUNKS = ROWS_PER_W // R
GROUPS = C // L


def _body(x_hbm, p_hbm, out_hbm, p_v, in_v, out_v):
    cid = lax.axis_index("c")
    sid = lax.axis_index("s")
    wid = sid * NC + cid
    base = wid * ROWS_PER_W * C

    pltpu.sync_copy(p_hbm, p_v)

    def chunk_body(ch, carry):
        off = base + ch * (R * C)
        pltpu.sync_copy(x_hbm.at[pl.ds(off, R * C)], in_v)

        def grp_body(g, c2):
            idx0 = p_v[pl.ds(g * L, L)]
            for r in range(R):
                vals = plsc.load_gather(in_v, [idx0 + r * C])
                out_v[pl.ds(r * C + g * L, L)] = vals
            return c2

        lax.fori_loop(0, GROUPS, grp_body, 0)
        pltpu.sync_copy(out_v, out_hbm.at[pl.ds(off, R * C)])
        return carry

    lax.fori_loop(0, CHUNKS, chunk_body, 0)


@jax.jit
def kernel(x, p):
    mesh = plsc.VectorSubcoreMesh(core_axis_name="c", subcore_axis_name="s")
    k = functools.partial(
        pl.kernel,
        out_type=jax.ShapeDtypeStruct((N_ROWS * C,), jnp.float32),
        mesh=mesh,
        scratch_types=[
            pltpu.VMEM((C,), jnp.int32),
            pltpu.VMEM((R * C,), jnp.float32),
            pltpu.VMEM((R * C,), jnp.float32),
        ],
        compiler_params=pltpu.CompilerParams(needs_layout_passes=False),
    )(_body)
    out_flat = k(x.reshape(-1), p.astype(jnp.int32))
    return out_flat.reshape(N_ROWS, C)


# speedup vs baseline: 1.1119x; 1.1119x over previous
import functools

import jax
import jax.numpy as jnp
from jax import lax
from jax.experimental import pallas as pl
from jax.experimental.pallas import tpu as pltpu
from jax.experimental.pallas import tpu_sc as plsc

N_ROWS = 8192
C = 2048
L = 16
NC = 2
NS = 16
NW = NC * NS
ROWS_PER_W = N_ROWS // NW
R = 8
CHUNKS = ROWS_PER_W // R
GROUPS = C // L


def _body(x_hbm, p_hbm, out_hbm, p_v,
          in0, in1, out0, out1, isem0, isem1, osem0, osem1):
    cid = lax.axis_index("c")
    sid = lax.axis_index("s")
    wid = sid * NC + cid
    base = wid * ROWS_PER_W * C

    pltpu.sync_copy(p_hbm, p_v)

    ins = (in0, in1)
    outs = (out0, out1)
    isems = (isem0, isem1)
    osems = (osem0, osem1)

    def in_copy(ch, b):
        off = base + ch * (R * C)
        return pltpu.async_copy(x_hbm.at[pl.ds(off, R * C)], ins[b], isems[b])

    def out_copy(ch, b):
        off = base + ch * (R * C)
        return pltpu.async_copy(outs[b], out_hbm.at[pl.ds(off, R * C)], osems[b])

    def permute(b):
        in_v = ins[b]
        out_v = outs[b]

        def grp_body(g, c2):
            idx0 = p_v[pl.ds(g * L, L)]
            for r in range(R):
                vals = plsc.load_gather(in_v, [idx0 + r * C])
                out_v[pl.ds(r * C + g * L, L)] = vals
            return c2

        lax.fori_loop(0, GROUPS, grp_body, 0)

    pending_in = [None, None]
    pending_out = [None, None]
    pending_in[0] = in_copy(0, 0)
    for ch in range(CHUNKS):
        b = ch % 2
        if ch + 1 < CHUNKS:
            pending_in[1 - b] = in_copy(ch + 1, 1 - b)
        pending_in[b].wait()
        if pending_out[b] is not None:
            pending_out[b].wait()
            pending_out[b] = None
        permute(b)
        pending_out[b] = out_copy(ch, b)
    for b in range(2):
        if pending_out[b] is not None:
            pending_out[b].wait()


@jax.jit
def kernel(x, p):
    mesh = plsc.VectorSubcoreMesh(core_axis_name="c", subcore_axis_name="s")
    k = functools.partial(
        pl.kernel,
        out_type=jax.ShapeDtypeStruct((N_ROWS * C,), jnp.float32),
        mesh=mesh,
        scratch_types=[
            pltpu.VMEM((C,), jnp.int32),
            pltpu.VMEM((R * C,), jnp.float32),
            pltpu.VMEM((R * C,), jnp.float32),
            pltpu.VMEM((R * C,), jnp.float32),
            pltpu.VMEM((R * C,), jnp.float32),
            pltpu.SemaphoreType.DMA,
            pltpu.SemaphoreType.DMA,
            pltpu.SemaphoreType.DMA,
            pltpu.SemaphoreType.DMA,
        ],
        compiler_params=pltpu.CompilerParams(needs_layout_passes=False),
    )(_body)
    out_flat = k(x.reshape(-1), p.astype(jnp.int32))
    return out_flat.reshape(N_ROWS, C)


# speedup vs baseline: 1.7202x; 1.5471x over previous
import functools

import jax
import jax.numpy as jnp
from jax import lax
from jax.experimental import pallas as pl
from jax.experimental.pallas import tpu as pltpu
from jax.experimental.pallas import tpu_sc as plsc

N_ROWS = 8192
C = 2048
L = 16
NC = 2
NS = 16
NW = NC * NS
ROWS_PER_W = N_ROWS // NW
R = 8
CHUNKS = ROWS_PER_W // R
GROUPS = C // L


def _body(x_hbm, p_hbm, out_hbm, p_v,
          in0, in1, out0, out1, isem0, isem1, osem0, osem1):
    cid = lax.axis_index("c")
    sid = lax.axis_index("s")
    wid = sid * NC + cid
    base = wid * ROWS_PER_W * C

    pltpu.sync_copy(p_hbm, p_v)

    ins = (in0, in1)
    outs = (out0, out1)
    isems = (isem0, isem1)
    osems = (osem0, osem1)

    def in_copy(ch, b):
        off = base + ch * (R * C)
        return pltpu.async_copy(x_hbm.at[pl.ds(off, R * C)], ins[b], isems[b])

    def out_copy(ch, b):
        off = base + ch * (R * C)
        return pltpu.async_copy(outs[b], out_hbm.at[pl.ds(off, R * C)], osems[b])

    def permute(b):
        in_v = ins[b]
        out_v = outs[b]

        @plsc.parallel_loop(0, GROUPS, unroll=4)
        def grp_body(g):
            idx0 = p_v[pl.ds(g * L, L)]
            for r in range(R):
                vals = plsc.load_gather(in_v, [idx0 + r * C])
                out_v[pl.ds(r * C + g * L, L)] = vals

    pending_in = [None, None]
    pending_out = [None, None]
    pending_in[0] = in_copy(0, 0)
    for ch in range(CHUNKS):
        b = ch % 2
        if ch + 1 < CHUNKS:
            pending_in[1 - b] = in_copy(ch + 1, 1 - b)
        pending_in[b].wait()
        if pending_out[b] is not None:
            pending_out[b].wait()
            pending_out[b] = None
        permute(b)
        pending_out[b] = out_copy(ch, b)
    for b in range(2):
        if pending_out[b] is not None:
            pending_out[b].wait()


@jax.jit
def kernel(x, p):
    mesh = plsc.VectorSubcoreMesh(core_axis_name="c", subcore_axis_name="s")
    k = functools.partial(
        pl.kernel,
        out_type=jax.ShapeDtypeStruct((N_ROWS * C,), jnp.float32),
        mesh=mesh,
        scratch_types=[
            pltpu.VMEM((C,), jnp.int32),
            pltpu.VMEM((R * C,), jnp.float32),
            pltpu.VMEM((R * C,), jnp.float32),
            pltpu.VMEM((R * C,), jnp.float32),
            pltpu.VMEM((R * C,), jnp.float32),
            pltpu.SemaphoreType.DMA,
            pltpu.SemaphoreType.DMA,
            pltpu.SemaphoreType.DMA,
            pltpu.SemaphoreType.DMA,
        ],
        compiler_params=pltpu.CompilerParams(needs_layout_passes=False),
    )(_body)
    out_flat = k(x.reshape(-1), p.astype(jnp.int32))
    return out_flat.reshape(N_ROWS, C)
